# initial kernel scaffold (unmeasured)
import jax
import jax.numpy as jnp
from jax import lax
from jax.experimental import pallas as pl
from jax.experimental.pallas import tpu as pltpu


def kernel(
    x,
):
    def body(*refs):
        pass

    out_shape = jax.ShapeDtypeStruct(..., jnp.float32)
    return pl.pallas_call(body, out_shape=out_shape)(...)


# baseline (device time: 19764 ns/iter reference)
import jax
import jax.numpy as jnp
from jax import lax
from jax.experimental import pallas as pl
from jax.experimental.pallas import tpu as pltpu

N_DEV = 4
M = 512
N_PER = 512


def kernel(x):
    def body(x_ref, out_ref, xbf, recv_buf, send_sems, recv_sems, local_sem):
        my = lax.axis_index("i")

        barrier_sem = pltpu.get_barrier_semaphore()
        for p in range(1, N_DEV):
            pl.semaphore_signal(
                barrier_sem, inc=1,
                device_id=((my + p) % N_DEV,),
                device_id_type=pl.DeviceIdType.MESH,
            )
        pl.semaphore_wait(barrier_sem, N_DEV - 1)

        xbf[...] = x_ref[0].astype(jnp.bfloat16)

        rdmas = []
        for s in range(N_DEV - 1):
            dst = (my + 1 + s) % N_DEV
            rdma = pltpu.make_async_remote_copy(
                src_ref=xbf.at[:, pl.ds(dst * N_PER, N_PER)],
                dst_ref=recv_buf.at[2 - s],
                send_sem=send_sems.at[s],
                recv_sem=recv_sems.at[2 - s],
                device_id=(dst,),
                device_id_type=pl.DeviceIdType.MESH,
            )
            rdma.start()
            rdmas.append(rdma)

        own = pltpu.make_async_copy(
            xbf.at[:, pl.ds(my * N_PER, N_PER)], recv_buf.at[3], local_sem
        )
        own.start()
        own.wait()

        for rdma in rdmas:
            rdma.wait()

        acc = recv_buf[3].astype(jnp.float32)
        for r in range(N_DEV - 1):
            acc = acc + recv_buf[r].astype(jnp.float32)
        out_ref[...] = acc

    return pl.pallas_call(
        body,
        out_shape=jax.ShapeDtypeStruct((M, N_PER), jnp.float32),
        in_specs=[pl.BlockSpec(memory_space=pltpu.VMEM)],
        out_specs=pl.BlockSpec(memory_space=pltpu.VMEM),
        scratch_shapes=[
            pltpu.VMEM((M, N_DEV * N_PER), jnp.bfloat16),
            pltpu.VMEM((N_DEV, M, N_PER), jnp.bfloat16),
            pltpu.SemaphoreType.DMA((N_DEV - 1,)),
            pltpu.SemaphoreType.DMA((N_DEV - 1,)),
            pltpu.SemaphoreType.DMA,
        ],
        compiler_params=pltpu.CompilerParams(collective_id=0),
    )(x)


# device time: 16930 ns/iter; 1.1674x vs baseline; 1.1674x over previous
import jax
import jax.numpy as jnp
from jax import lax
from jax.experimental import pallas as pl
from jax.experimental.pallas import tpu as pltpu

N_DEV = 4
M = 512
H = M // 2
N_PER = 512

FWDA, FWDB, DIRA, DIRB, SUMA, SUMB = range(6)


def kernel(x):
    def body(x_ref, out_ref, xbf, rbuf, sA, sB, own0, ownL, ownR,
             send_sems, recv_sems, local_sems):
        my = lax.axis_index("i")
        left = (my - 1) % N_DEV
        right = (my + 1) % N_DEV
        opp = (my + 2) % N_DEV

        barrier_sem = pltpu.get_barrier_semaphore()
        for nbr in [left, right]:
            pl.semaphore_signal(
                barrier_sem, inc=1,
                device_id=(nbr,), device_id_type=pl.DeviceIdType.MESH,
            )
        pl.semaphore_wait(barrier_sem, 2)

        xbf[...] = x_ref[0].astype(jnp.bfloat16)

        def col(ref, c, r0, rn):
            return ref.at[pl.ds(r0, rn), pl.ds(c * N_PER, N_PER)]

        def remote(src, dst, sem_ix, target):
            return pltpu.make_async_remote_copy(
                src_ref=src, dst_ref=dst,
                send_sem=send_sems.at[sem_ix], recv_sem=recv_sems.at[sem_ix],
                device_id=(target,), device_id_type=pl.DeviceIdType.MESH,
            )

        fwdA = remote(col(xbf, opp, 0, H), rbuf.at[FWDA], FWDA, left)
        fwdB = remote(col(xbf, opp, H, H), rbuf.at[FWDB], FWDB, right)
        dirA = remote(col(xbf, right, 0, H), rbuf.at[DIRA], DIRA, right)
        dirB = remote(col(xbf, left, H, H), rbuf.at[DIRB], DIRB, left)
        fwdA.start()
        fwdB.start()
        dirA.start()
        dirB.start()

        cp0 = pltpu.make_async_copy(col(xbf, my, 0, M), own0, local_sems.at[0])
        cpL = pltpu.make_async_copy(col(xbf, left, 0, H), ownL, local_sems.at[1])
        cpR = pltpu.make_async_copy(col(xbf, right, H, H), ownR, local_sems.at[2])
        cp0.start()
        cpL.start()
        cpR.start()

        fwdA.wait_recv()
        cpL.wait()
        sA[...] = rbuf[FWDA] + ownL[...]
        sumA = remote(sA.at[:, :], rbuf.at[SUMA], SUMA, left)
        sumA.start()

        fwdB.wait_recv()
        cpR.wait()
        sB[...] = rbuf[FWDB] + ownR[...]
        sumB = remote(sB.at[:, :], rbuf.at[SUMB], SUMB, right)
        sumB.start()

        cp0.wait()
        dirA.wait_recv()
        sumA.wait_recv()
        out_ref[0:H, :] = (
            own0[0:H, :].astype(jnp.float32)
            + rbuf[DIRA].astype(jnp.float32)
            + rbuf[SUMA].astype(jnp.float32)
        )
        dirB.wait_recv()
        sumB.wait_recv()
        out_ref[H:M, :] = (
            own0[H:M, :].astype(jnp.float32)
            + rbuf[DIRB].astype(jnp.float32)
            + rbuf[SUMB].astype(jnp.float32)
        )

        for r in (fwdA, fwdB, dirA, dirB, sumA, sumB):
            r.wait_send()

    return pl.pallas_call(
        body,
        out_shape=jax.ShapeDtypeStruct((M, N_PER), jnp.float32),
        in_specs=[pl.BlockSpec(memory_space=pltpu.VMEM)],
        out_specs=pl.BlockSpec(memory_space=pltpu.VMEM),
        scratch_shapes=[
            pltpu.VMEM((M, N_DEV * N_PER), jnp.bfloat16),
            pltpu.VMEM((6, H, N_PER), jnp.bfloat16),
            pltpu.VMEM((H, N_PER), jnp.bfloat16),
            pltpu.VMEM((H, N_PER), jnp.bfloat16),
            pltpu.VMEM((M, N_PER), jnp.bfloat16),
            pltpu.VMEM((H, N_PER), jnp.bfloat16),
            pltpu.VMEM((H, N_PER), jnp.bfloat16),
            pltpu.SemaphoreType.DMA((6,)),
            pltpu.SemaphoreType.DMA((6,)),
            pltpu.SemaphoreType.DMA((3,)),
        ],
        compiler_params=pltpu.CompilerParams(collective_id=0),
    )(x)


# device time: 16711 ns/iter; 1.1827x vs baseline; 1.0131x over previous
import jax
import jax.numpy as jnp
from jax import lax
from jax.experimental import pallas as pl
from jax.experimental.pallas import tpu as pltpu

N_DEV = 4
M = 512
H = M // 2
N_PER = 512

FWDA, FWDB, DIRA, DIRB, SUMA, SUMB = range(6)


def kernel(x):
    def body(x_ref, out_ref, sOppA, sOppB, sDirA, sDirB, sA, sB, rbuf,
             send_sems, recv_sems):
        my = lax.axis_index("i")
        left = (my - 1) % N_DEV
        right = (my + 1) % N_DEV
        opp = (my + 2) % N_DEV

        barrier_sem = pltpu.get_barrier_semaphore()
        for nbr in [left, right]:
            pl.semaphore_signal(
                barrier_sem, inc=1,
                device_id=(nbr,), device_id_type=pl.DeviceIdType.MESH,
            )
        pl.semaphore_wait(barrier_sem, 2)

        def remote(src, dst_ix, target):
            return pltpu.make_async_remote_copy(
                src_ref=src, dst_ref=rbuf.at[dst_ix],
                send_sem=send_sems.at[dst_ix], recv_sem=recv_sems.at[dst_ix],
                device_id=(target,), device_id_type=pl.DeviceIdType.MESH,
            )

        fwdA = remote(sOppA.at[:, :], FWDA, left)
        fwdB = remote(sOppB.at[:, :], FWDB, right)
        dirA = remote(sDirA.at[:, :], DIRA, right)
        dirB = remote(sDirB.at[:, :], DIRB, left)
        sumA = remote(sA.at[:, :], SUMA, left)
        sumB = remote(sB.at[:, :], SUMB, right)

        def chunk(c, r0, rn):
            return x_ref[0, r0:r0 + rn, c * N_PER:(c + 1) * N_PER]

        for c in range(N_DEV):
            @pl.when(opp == c)
            def _(c=c):
                sOppA[...] = chunk(c, 0, H).astype(jnp.bfloat16)
                sOppB[...] = chunk(c, H, H).astype(jnp.bfloat16)
        fwdA.start()
        fwdB.start()

        for c in range(N_DEV):
            @pl.when(right == c)
            def _(c=c):
                sDirA[...] = chunk(c, 0, H).astype(jnp.bfloat16)
        dirA.start()
        for c in range(N_DEV):
            @pl.when(left == c)
            def _(c=c):
                sDirB[...] = chunk(c, H, H).astype(jnp.bfloat16)
        dirB.start()

        for c in range(N_DEV):
            @pl.when(my == c)
            def _(c=c):
                out_ref[...] = chunk(c, 0, M)

        fwdA.wait_recv()
        for c in range(N_DEV):
            @pl.when(left == c)
            def _(c=c):
                sA[...] = rbuf[FWDA] + chunk(c, 0, H).astype(jnp.bfloat16)
        sumA.start()

        fwdB.wait_recv()
        for c in range(N_DEV):
            @pl.when(right == c)
            def _(c=c):
                sB[...] = rbuf[FWDB] + chunk(c, H, H).astype(jnp.bfloat16)
        sumB.start()

        dirA.wait_recv()
        sumA.wait_recv()
        out_ref[0:H, :] = (
            out_ref[0:H, :]
            + rbuf[DIRA].astype(jnp.float32)
            + rbuf[SUMA].astype(jnp.float32)
        )
        dirB.wait_recv()
        sumB.wait_recv()
        out_ref[H:M, :] = (
            out_ref[H:M, :]
            + rbuf[DIRB].astype(jnp.float32)
            + rbuf[SUMB].astype(jnp.float32)
        )

        for r in (fwdA, fwdB, dirA, dirB, sumA, sumB):
            r.wait_send()

    return pl.pallas_call(
        body,
        out_shape=jax.ShapeDtypeStruct((M, N_PER), jnp.float32),
        in_specs=[pl.BlockSpec(memory_space=pltpu.VMEM)],
        out_specs=pl.BlockSpec(memory_space=pltpu.VMEM),
        scratch_shapes=[
            pltpu.VMEM((H, N_PER), jnp.bfloat16),
            pltpu.VMEM((H, N_PER), jnp.bfloat16),
            pltpu.VMEM((H, N_PER), jnp.bfloat16),
            pltpu.VMEM((H, N_PER), jnp.bfloat16),
            pltpu.VMEM((H, N_PER), jnp.bfloat16),
            pltpu.VMEM((H, N_PER), jnp.bfloat16),
            pltpu.VMEM((6, H, N_PER), jnp.bfloat16),
            pltpu.SemaphoreType.DMA((6,)),
            pltpu.SemaphoreType.DMA((6,)),
        ],
        compiler_params=pltpu.CompilerParams(collective_id=0),
    )(x)
